# baseline (speedup 1.0000x reference)
import jax
import jax.numpy as jnp
from jax.experimental import pallas as pl
from jax.experimental.pallas import tpu as pltpu

_DIM = 1024
_INTER = 512
_E = 8
_TOPK = 2
_SHINTER = 1024
_KALL = _E * _INTER + _SHINTER
_NT = 2


def _dot_t(a, b, prec=None):
    return jax.lax.dot_general(
        a, b, (((1,), (1,)), ((), ())),
        preferred_element_type=jnp.float32, precision=prec)


def _moe_body(T, xb_ref, wg_ref, w13_ref, ws13_ref, bs13_ref, w2all_ref,
              bs2_ref, y_ref, l_ref, w_scr, h_scr, cnt_scr, prb_scr):
    t = pl.program_id(0)
    e = pl.program_id(1)
    nt = pl.num_programs(0)
    TT = xb_ref.shape[0]

    @pl.when(e == 0)
    def _gate():
        scores = _dot_t(xb_ref[...], wg_ref[...])
        p = jax.nn.sigmoid(scores)
        iota = jax.lax.broadcasted_iota(jnp.int32, p.shape, 1)
        m1 = jnp.max(p, axis=1, keepdims=True)
        am1 = jnp.min(jnp.where(p == m1, iota, _E), axis=1, keepdims=True)
        p2 = jnp.where(iota == am1, -1.0, p)
        m2 = jnp.max(p2, axis=1, keepdims=True)
        am2 = jnp.min(jnp.where(p2 == m2, iota, _E), axis=1, keepdims=True)
        s = m1 + m2
        w = (jnp.where(iota == am1, m1, 0.0) +
             jnp.where(iota == am2, m2, 0.0)) / s
        w_scr[...] = w
        sel = ((iota == am1) | (iota == am2)).astype(jnp.float32)
        counts = jnp.sum(sel, axis=0, keepdims=True)
        probs = jnp.sum(w, axis=0, keepdims=True)

        @pl.when(t == 0)
        def _():
            cnt_scr[...] = counts
            prb_scr[...] = probs

        @pl.when(t > 0)
        def _():
            cnt_scr[...] += counts
            prb_scr[...] += probs

    @pl.when(e < _E)
    def _routed():
        g = _dot_t(xb_ref[...], w13_ref[0])
        iota = jax.lax.broadcasted_iota(jnp.int32, (TT, _E), 1)
        wtok = jnp.sum(jnp.where(iota == e, w_scr[...], 0.0),
                       axis=1, keepdims=True)
        h = jax.nn.silu(g[:, :_INTER]) * g[:, _INTER:] * wtok
        h_scr[:, pl.ds(e * _INTER, _INTER)] = h.astype(jnp.bfloat16)

    @pl.when(e == _E)
    def _shared():
        gs = _dot_t(xb_ref[...], ws13_ref[...]) + bs13_ref[...]
        hs = jax.nn.silu(gs[:, :_SHINTER]) * gs[:, _SHINTER:]
        h_scr[:, _E * _INTER:] = hs.astype(jnp.bfloat16)
        y_ref[...] = _dot_t(h_scr[...], w2all_ref[...]) + bs2_ref[...]

        @pl.when(t == nt - 1)
        def _():
            f_i = _E * cnt_scr[...] / (_TOPK * T)
            p_i = prb_scr[...] / T
            l_ref[...] = jnp.sum(f_i * p_i, axis=1, keepdims=True)


def kernel(x, Wg, W1, W2, W3, Ws1, bs1, Ws2, bs2, Ws3, bs3):
    import functools
    orig_shape = x.shape
    xf = x.reshape(-1, _DIM)
    T = xf.shape[0]
    TT = T // _NT
    bf = jnp.bfloat16
    xb = xf.astype(bf)
    Wgb = Wg.astype(bf)
    W13 = jnp.concatenate([W1, W3], axis=1).astype(bf)
    Ws13 = jnp.concatenate([Ws1, Ws3], axis=0).astype(bf)
    bs13 = jnp.concatenate([bs1, bs3]).reshape(1, 2 * _SHINTER)
    W2all = jnp.concatenate(
        [W2.transpose(1, 0, 2).reshape(_DIM, _E * _INTER), Ws2], axis=1
    ).astype(bf)
    bs2r = bs2.reshape(1, _DIM)

    const2 = lambda shape: pl.BlockSpec(shape, lambda t, e: (0, 0))
    expert3 = lambda shape: pl.BlockSpec(
        shape, lambda t, e: (jnp.minimum(e, _E - 1), 0, 0))
    tok2 = lambda shape: pl.BlockSpec(shape, lambda t, e: (t, 0))

    y, l = pl.pallas_call(
        functools.partial(_moe_body, T),
        grid=(_NT, _E + 1),
        in_specs=[
            tok2((TT, _DIM)),
            const2((_E, _DIM)),
            expert3((1, 2 * _INTER, _DIM)),
            const2((2 * _SHINTER, _DIM)),
            const2((1, 2 * _SHINTER)),
            const2((_DIM, _KALL)),
            const2((1, _DIM)),
        ],
        out_specs=[
            tok2((TT, _DIM)),
            const2((1, 1)),
        ],
        out_shape=[
            jax.ShapeDtypeStruct((T, _DIM), jnp.float32),
            jax.ShapeDtypeStruct((1, 1), jnp.float32),
        ],
        scratch_shapes=[
            pltpu.VMEM((TT, _E), jnp.float32),
            pltpu.VMEM((TT, _KALL), bf),
            pltpu.VMEM((1, _E), jnp.float32),
            pltpu.VMEM((1, _E), jnp.float32),
        ],
        compiler_params=pltpu.CompilerParams(
            dimension_semantics=("arbitrary", "arbitrary")),
    )(xb, Wgb, W13, Ws13, bs13, W2all, bs2r)
    return y.reshape(orig_shape), l[0, 0]
